# baseline (speedup 1.0000x reference)
import functools

import jax
import jax.numpy as jnp
from jax.experimental import pallas as pl
from jax.experimental.pallas import tpu as pltpu

HIDDEN_DIM = 768
EXPERT_NUMBER = 64
TOP_K = 2
BLOCK_N = 2048

_NEG_INF = float("-inf")


def _router_kernel(x_ref, w_ref, b_ref, logits_t_ref, weights_t_ref,
                   idx_t_ref, mask_ref):
    logits_t = jax.lax.dot_general(
        w_ref[...], x_ref[...], (((1,), (1,)), ((), ())),
        preferred_element_type=jnp.float32)
    logits_t = logits_t + b_ref[...]
    logits_t_ref[...] = logits_t

    e_iota = jax.lax.broadcasted_iota(jnp.int32, logits_t.shape, 0)
    m1 = jnp.max(logits_t, axis=0)
    oh1 = logits_t == m1[None, :]
    i1 = jnp.min(jnp.where(oh1, e_iota, EXPERT_NUMBER), axis=0)
    oh1 = e_iota == i1[None, :]
    masked = jnp.where(oh1, _NEG_INF, logits_t)
    m2 = jnp.max(masked, axis=0)
    oh2 = masked == m2[None, :]
    i2 = jnp.min(jnp.where(oh2, e_iota, EXPERT_NUMBER), axis=0)
    oh2 = e_iota == i2[None, :]

    w1 = jax.nn.sigmoid(m1 - m2)
    weights_t_ref[...] = jnp.stack([w1, 1.0 - w1], axis=0)
    idx_t_ref[...] = jnp.stack([i1, i2], axis=0)

    bn = logits_t.shape[1]
    mask_iota = jax.lax.broadcasted_iota(jnp.int32, (EXPERT_NUMBER, TOP_K, bn),
                                         0)
    sel = jnp.stack([i1, i2], axis=0)
    mask_ref[...] = (mask_iota == sel[None, :, :]).astype(jnp.int32)


@functools.partial(jax.jit, static_argnames=())
def kernel(x, W, b):
    n_tokens = x.shape[0]
    grid = (n_tokens // BLOCK_N,)
    b_col = b.reshape(EXPERT_NUMBER, 1)
    out_types = (
        jax.ShapeDtypeStruct((EXPERT_NUMBER, n_tokens), jnp.float32),
        jax.ShapeDtypeStruct((TOP_K, n_tokens), jnp.float32),
        jax.ShapeDtypeStruct((TOP_K, n_tokens), jnp.int32),
        jax.ShapeDtypeStruct((EXPERT_NUMBER, TOP_K, n_tokens), jnp.int32),
    )
    logits_t, weights_t, idx_t, mask = pl.pallas_call(
        _router_kernel,
        grid=grid,
        in_specs=[
            pl.BlockSpec((BLOCK_N, HIDDEN_DIM), lambda i: (i, 0)),
            pl.BlockSpec((EXPERT_NUMBER, HIDDEN_DIM), lambda i: (0, 0)),
            pl.BlockSpec((EXPERT_NUMBER, 1), lambda i: (0, 0)),
        ],
        out_specs=[
            pl.BlockSpec((EXPERT_NUMBER, BLOCK_N), lambda i: (0, i)),
            pl.BlockSpec((TOP_K, BLOCK_N), lambda i: (0, i)),
            pl.BlockSpec((TOP_K, BLOCK_N), lambda i: (0, i)),
            pl.BlockSpec((EXPERT_NUMBER, TOP_K, BLOCK_N), lambda i: (0, 0, i)),
        ],
        out_shape=out_types,
        compiler_params=pltpu.CompilerParams(
            dimension_semantics=("parallel",)),
    )(x, W, b_col)
    return (logits_t.T, weights_t.T, idx_t.T, mask)


# speedup vs baseline: 1.0506x; 1.0506x over previous
import functools

import jax
import jax.numpy as jnp
from jax.experimental import pallas as pl
from jax.experimental.pallas import tpu as pltpu

HIDDEN_DIM = 768
EXPERT_NUMBER = 64
TOP_K = 2
BLOCK_N = 4096

_NEG_INF = float("-inf")


def _router_kernel(x_ref, w_ref, b_ref, logits_t_ref, weights_t_ref,
                   idx_t_ref, mask_ref):
    logits_t = jax.lax.dot_general(
        w_ref[...], x_ref[...], (((1,), (1,)), ((), ())),
        preferred_element_type=jnp.float32)
    logits_t = logits_t + b_ref[...]
    logits_t_ref[...] = logits_t

    e_iota = jax.lax.broadcasted_iota(jnp.int32, logits_t.shape, 0)
    m1 = jnp.max(logits_t, axis=0)
    oh1 = logits_t == m1[None, :]
    i1 = jnp.min(jnp.where(oh1, e_iota, EXPERT_NUMBER), axis=0)
    oh1 = e_iota == i1[None, :]
    masked = jnp.where(oh1, _NEG_INF, logits_t)
    m2 = jnp.max(masked, axis=0)
    oh2 = masked == m2[None, :]
    i2 = jnp.min(jnp.where(oh2, e_iota, EXPERT_NUMBER), axis=0)
    oh2 = e_iota == i2[None, :]

    w1 = jax.nn.sigmoid(m1 - m2)
    weights_t_ref[...] = jnp.stack([w1, 1.0 - w1], axis=0)
    idx_t_ref[...] = jnp.stack([i1, i2], axis=0)

    bn = logits_t.shape[1]
    mask_iota = jax.lax.broadcasted_iota(jnp.int32, (EXPERT_NUMBER, TOP_K, bn),
                                         0)
    sel = jnp.stack([i1, i2], axis=0)
    mask_ref[...] = (mask_iota == sel[None, :, :]).astype(jnp.int32)


@functools.partial(jax.jit, static_argnames=())
def kernel(x, W, b):
    n_tokens = x.shape[0]
    grid = (n_tokens // BLOCK_N,)
    b_col = b.reshape(EXPERT_NUMBER, 1)
    out_types = (
        jax.ShapeDtypeStruct((EXPERT_NUMBER, n_tokens), jnp.float32),
        jax.ShapeDtypeStruct((TOP_K, n_tokens), jnp.float32),
        jax.ShapeDtypeStruct((TOP_K, n_tokens), jnp.int32),
        jax.ShapeDtypeStruct((EXPERT_NUMBER, TOP_K, n_tokens), jnp.int32),
    )
    logits_t, weights_t, idx_t, mask = pl.pallas_call(
        _router_kernel,
        grid=grid,
        in_specs=[
            pl.BlockSpec((BLOCK_N, HIDDEN_DIM), lambda i: (i, 0)),
            pl.BlockSpec((EXPERT_NUMBER, HIDDEN_DIM), lambda i: (0, 0)),
            pl.BlockSpec((EXPERT_NUMBER, 1), lambda i: (0, 0)),
        ],
        out_specs=[
            pl.BlockSpec((EXPERT_NUMBER, BLOCK_N), lambda i: (0, i)),
            pl.BlockSpec((TOP_K, BLOCK_N), lambda i: (0, i)),
            pl.BlockSpec((TOP_K, BLOCK_N), lambda i: (0, i)),
            pl.BlockSpec((EXPERT_NUMBER, TOP_K, BLOCK_N), lambda i: (0, 0, i)),
        ],
        out_shape=out_types,
        compiler_params=pltpu.CompilerParams(
            dimension_semantics=("parallel",)),
    )(x, W, b_col)
    return (logits_t.T, weights_t.T, idx_t.T, mask)


# speedup vs baseline: 1.0820x; 1.0300x over previous
import functools

import jax
import jax.numpy as jnp
from jax.experimental import pallas as pl
from jax.experimental.pallas import tpu as pltpu

HIDDEN_DIM = 768
EXPERT_NUMBER = 64
TOP_K = 2
BLOCK_N = 4096

_NEG_INF = float("-inf")


def _router_kernel(x_ref, w_ref, b_ref, logits_t_ref, weights_t_ref,
                   idx_t_ref, mask_ref):
    logits_t = jax.lax.dot_general(
        w_ref[...], x_ref[...], (((1,), (1,)), ((), ())),
        preferred_element_type=jnp.float32)
    logits_t = logits_t + b_ref[...][:, None]
    logits_t_ref[...] = logits_t

    e_iota = jax.lax.broadcasted_iota(jnp.int32, logits_t.shape, 0)
    m1 = jnp.max(logits_t, axis=0)
    oh1 = logits_t == m1[None, :]
    i1 = jnp.min(jnp.where(oh1, e_iota, EXPERT_NUMBER), axis=0)
    oh1 = e_iota == i1[None, :]
    masked = jnp.where(oh1, _NEG_INF, logits_t)
    m2 = jnp.max(masked, axis=0)
    oh2 = masked == m2[None, :]
    i2 = jnp.min(jnp.where(oh2, e_iota, EXPERT_NUMBER), axis=0)
    oh2 = e_iota == i2[None, :]

    w1 = jax.nn.sigmoid(m1 - m2)
    weights_t_ref[...] = jnp.stack([w1, 1.0 - w1], axis=0)
    idx_t_ref[...] = jnp.stack([i1, i2], axis=0)

    bn = logits_t.shape[1]
    mask_iota = jax.lax.broadcasted_iota(jnp.int32, (EXPERT_NUMBER, TOP_K, bn),
                                         0)
    sel = jnp.stack([i1, i2], axis=0)
    mask_ref[...] = (mask_iota == sel[None, :, :]).astype(jnp.int32)


@functools.partial(jax.jit, static_argnames=())
def kernel(x, W, b):
    n_tokens = x.shape[0]
    grid = (n_tokens // BLOCK_N,)

    out_types = (
        jax.ShapeDtypeStruct((EXPERT_NUMBER, n_tokens), jnp.float32),
        jax.ShapeDtypeStruct((TOP_K, n_tokens), jnp.float32),
        jax.ShapeDtypeStruct((TOP_K, n_tokens), jnp.int32),
        jax.ShapeDtypeStruct((EXPERT_NUMBER, TOP_K, n_tokens), jnp.int32),
    )
    logits_t, weights_t, idx_t, mask = pl.pallas_call(
        _router_kernel,
        grid=grid,
        in_specs=[
            pl.BlockSpec((BLOCK_N, HIDDEN_DIM), lambda i: (i, 0)),
            pl.BlockSpec((EXPERT_NUMBER, HIDDEN_DIM), lambda i: (0, 0)),
            pl.BlockSpec((EXPERT_NUMBER,), lambda i: (0,)),
        ],
        out_specs=[
            pl.BlockSpec((EXPERT_NUMBER, BLOCK_N), lambda i: (0, i)),
            pl.BlockSpec((TOP_K, BLOCK_N), lambda i: (0, i)),
            pl.BlockSpec((TOP_K, BLOCK_N), lambda i: (0, i)),
            pl.BlockSpec((EXPERT_NUMBER, TOP_K, BLOCK_N), lambda i: (0, 0, i)),
        ],
        out_shape=out_types,
        compiler_params=pltpu.CompilerParams(
            dimension_semantics=("parallel",)),
    )(x, W, b)
    return (logits_t.T, weights_t.T, idx_t.T, mask)
